# Initial kernel scaffold; baseline (speedup 1.0000x reference)
import jax
import jax.numpy as jnp
from jax.experimental import pallas as pl


def kernel(x, emb_weight):
    raise NotImplementedError("write your pallas kernel here")


# speedup vs baseline: 2.9731x; 2.9731x over previous
import functools

import jax
import jax.numpy as jnp
from jax import lax
from jax.experimental import pallas as pl
from jax.experimental.pallas import tpu as pltpu
from jax.experimental.pallas import tpu_sc as plsc

VOCAB = 100000
DIM = 128
NC = 2
NS = 16
NW = NC * NS
CHUNK = 128
NCHUNK = 50


def _body(x_hbm, tbl_hbm, out_hbm, idx_v, rows_v, gsem):
    wid = lax.axis_index("s") * NC + lax.axis_index("c")
    pltpu.sync_copy(x_hbm.at[wid], idx_v)

    def step(j, carry):
        pltpu.async_copy(tbl_hbm.at[idx_v.at[j]], rows_v, gsem).wait()
        pltpu.sync_copy(rows_v, out_hbm.at[wid, j])
        return carry

    lax.fori_loop(0, NCHUNK, step, 0)


@functools.partial(jax.jit, static_argnames=())
def _run(x_flat, emb_weight):
    mesh = plsc.VectorSubcoreMesh(core_axis_name="c", subcore_axis_name="s")
    k = pl.kernel(
        _body,
        out_type=jax.ShapeDtypeStruct((NW, NCHUNK, CHUNK, DIM), jnp.float32),
        mesh=mesh,
        scratch_types=[
            pltpu.VMEM((NCHUNK, CHUNK), jnp.int32),
            pltpu.VMEM((CHUNK, DIM), jnp.float32),
            pltpu.SemaphoreType.DMA,
        ],
    )
    return k(x_flat, emb_weight)


def kernel(x, emb_weight):
    b, s = x.shape
    x_flat = x.reshape(NW, NCHUNK, CHUNK).astype(jnp.int32)
    out = _run(x_flat, emb_weight)
    return out.reshape(b, s, DIM)


# speedup vs baseline: 3.3196x; 1.1166x over previous
import functools

import jax
import jax.numpy as jnp
from jax import lax
from jax.experimental import pallas as pl
from jax.experimental.pallas import tpu as pltpu
from jax.experimental.pallas import tpu_sc as plsc

VOCAB = 100000
DIM = 128
NC = 2
NS = 16
NW = NC * NS
CHUNK = 128
NCHUNK = 50


NBUF = 5
NGROUP = NCHUNK // NBUF


def _body(x_hbm, tbl_hbm, out_hbm, idx_v, rows_v, gsem, osem):
    wid = lax.axis_index("s") * NC + lax.axis_index("c")
    pltpu.sync_copy(x_hbm.at[wid], idx_v)

    def start_gather(j, b):
        pltpu.async_copy(tbl_hbm.at[idx_v.at[j]], rows_v.at[b], gsem.at[b])

    def wait_gather(b):
        pltpu.make_async_copy(
            tbl_hbm.at[idx_v.at[0]], rows_v.at[b], gsem.at[b]).wait()

    def start_out(j, b):
        pltpu.async_copy(rows_v.at[b], out_hbm.at[wid, j], osem.at[b])

    def wait_out(b):
        pltpu.make_async_copy(
            rows_v.at[b], out_hbm.at[wid, 0], osem.at[b]).wait()

    for b in range(NBUF):
        start_gather(b, b)

    def group(g, carry):
        for b in range(NBUF):
            wait_gather(b)
            start_out(g * NBUF + b, b)
        for b in range(NBUF):
            wait_out(b)

            @pl.when(g + 1 < NGROUP)
            def _():
                start_gather((g + 1) * NBUF + b, b)

        return carry

    lax.fori_loop(0, NGROUP, group, 0)


@functools.partial(jax.jit, static_argnames=())
def _run(x_flat, emb_weight):
    mesh = plsc.VectorSubcoreMesh(core_axis_name="c", subcore_axis_name="s")
    k = pl.kernel(
        _body,
        out_type=jax.ShapeDtypeStruct((NW, NCHUNK, CHUNK, DIM), jnp.float32),
        mesh=mesh,
        scratch_types=[
            pltpu.VMEM((NCHUNK, CHUNK), jnp.int32),
            pltpu.VMEM((NBUF, CHUNK, DIM), jnp.float32),
            pltpu.SemaphoreType.DMA((NBUF,)),
            pltpu.SemaphoreType.DMA((NBUF,)),
        ],
    )
    return k(x_flat, emb_weight)


def kernel(x, emb_weight):
    b, s = x.shape
    x_flat = x.reshape(NW, NCHUNK, CHUNK).astype(jnp.int32)
    out = _run(x_flat, emb_weight)
    return out.reshape(b, s, DIM)
